# baseline (speedup 1.0000x reference)
import functools

import jax
import jax.numpy as jnp
from jax import lax
from jax.experimental import pallas as pl
from jax.experimental.pallas import tpu as pltpu
from jax.experimental.pallas import tpu_sc as plsc

B, SEQ, D = 32, 1, 1024
E_ROUTED, E_SHARED, H, TOP_K = 64, 2, 512, 2
T = B * SEQ
NEG = -3.0e38


def _router_body(x_ref, rw_ref, logits_ref, logits_t_ref):
    logits_ref[...] = jnp.dot(x_ref[...], rw_ref[...],
                              preferred_element_type=jnp.float32)
    logits_t_ref[...] = lax.dot_general(
        rw_ref[...], x_ref[...], (((0,), (1,)), ((), ())),
        preferred_element_type=jnp.float32)


def _routing_sc_body(lt_hbm, gdt_hbm, idx_hbm, gates_hbm, sched_hbm, nact_hbm,
                     lt_v, gdt_v, idx_v, gates_v, sched_v, nact_v):
    wid = lax.axis_index("s") * 2 + lax.axis_index("c")

    @pl.when(wid == 0)
    def _():
        pltpu.sync_copy(lt_hbm, lt_v)

        iota = lax.iota(jnp.int32, 16)
        zf = jnp.zeros((16,), jnp.float32)
        zi = jnp.zeros((16,), jnp.int32)

        def top2_step(e, c):
            m1a, i1a, m2a, i2a, m1b, i1b, m2b, i2b = c
            ev = jnp.full((16,), e, jnp.int32)
            va = lt_v[e, pl.ds(0, 16)]
            vb = lt_v[e, pl.ds(16, 16)]
            g1 = va > m1a
            g2 = va > m2a
            m2a = jnp.where(g1, m1a, jnp.where(g2, va, m2a))
            i2a = jnp.where(g1, i1a, jnp.where(g2, ev, i2a))
            m1a = jnp.where(g1, va, m1a)
            i1a = jnp.where(g1, ev, i1a)
            g1 = vb > m1b
            g2 = vb > m2b
            m2b = jnp.where(g1, m1b, jnp.where(g2, vb, m2b))
            i2b = jnp.where(g1, i1b, jnp.where(g2, ev, i2b))
            m1b = jnp.where(g1, vb, m1b)
            i1b = jnp.where(g1, ev, i1b)
            return (m1a, i1a, m2a, i2a, m1b, i1b, m2b, i2b)

        neg = jnp.full((16,), NEG, jnp.float32)
        m1a, i1a, m2a, i2a, m1b, i1b, m2b, i2b = lax.fori_loop(
            0, E_ROUTED, top2_step, (neg, zi, neg, zi, neg, zi, neg, zi))

        def den_step(e, c):
            sa, sb = c
            va = lt_v[e, pl.ds(0, 16)]
            vb = lt_v[e, pl.ds(16, 16)]
            return sa + jnp.exp(va - m1a), sb + jnp.exp(vb - m1b)

        sa, sb = lax.fori_loop(0, E_ROUTED, den_step, (zf, zf))
        g1a = 1.0 / sa
        g1b = 1.0 / sb
        g2a = jnp.exp(m2a - m1a) / sa
        g2b = jnp.exp(m2b - m1b) / sb

        idx_v[0, pl.ds(0, 16)] = i1a
        idx_v[0, pl.ds(16, 16)] = i1b
        idx_v[1, pl.ds(0, 16)] = i2a
        idx_v[1, pl.ds(16, 16)] = i2b
        gates_v[0, pl.ds(0, 16)] = g1a
        gates_v[0, pl.ds(16, 16)] = g1b
        gates_v[1, pl.ds(0, 16)] = g2a
        gates_v[1, pl.ds(16, 16)] = g2b

        def expert_step(e, c):
            cnt, last, s0, s1, s2, s3 = c
            ev = jnp.full((16,), e, jnp.int32)
            hit1a = i1a == ev
            hit2a = i2a == ev
            hit1b = i1b == ev
            hit2b = i2b == ev
            ra = jnp.where(hit1a, g1a, 0.0) + jnp.where(hit2a, g2a, 0.0)
            rb = jnp.where(hit1b, g1b, 0.0) + jnp.where(hit2b, g2b, 0.0)
            gdt_v[e, pl.ds(0, 16)] = ra
            gdt_v[e, pl.ds(16, 16)] = rb
            na = (jnp.sum(jnp.where(hit1a | hit2a, 1, 0))
                  + jnp.sum(jnp.where(hit1b | hit2b, 1, 0)))
            active = na > 0
            target = jnp.where(active, cnt, -1)
            s0 = jnp.where(iota == target, e, s0)
            s1 = jnp.where(iota + 16 == target, e, s1)
            s2 = jnp.where(iota + 32 == target, e, s2)
            s3 = jnp.where(iota + 48 == target, e, s3)
            cnt = jnp.where(active, cnt + 1, cnt)
            last = jnp.where(active, e, last)
            return (cnt, last, s0, s1, s2, s3)

        cnt, last, s0, s1, s2, s3 = lax.fori_loop(
            0, E_ROUTED, expert_step,
            (jnp.int32(0), jnp.int32(0), zi, zi, zi, zi))

        lastv = jnp.full((16,), last, jnp.int32)
        sched_v[pl.ds(0, 16)] = jnp.where(iota < cnt, s0, lastv)
        sched_v[pl.ds(16, 16)] = jnp.where(iota + 16 < cnt, s1, lastv)
        sched_v[pl.ds(32, 16)] = jnp.where(iota + 32 < cnt, s2, lastv)
        sched_v[pl.ds(48, 16)] = jnp.where(iota + 48 < cnt, s3, lastv)
        nact_v[...] = jnp.where(iota == 0, cnt, 0)

        pltpu.sync_copy(gdt_v, gdt_hbm)
        pltpu.sync_copy(idx_v, idx_hbm)
        pltpu.sync_copy(gates_v, gates_hbm)
        pltpu.sync_copy(sched_v, sched_hbm)
        pltpu.sync_copy(nact_v, nact_hbm)


_routing_sc = functools.partial(
    pl.kernel,
    out_type=(
        jax.ShapeDtypeStruct((E_ROUTED, T), jnp.float32),
        jax.ShapeDtypeStruct((TOP_K, T), jnp.int32),
        jax.ShapeDtypeStruct((TOP_K, T), jnp.float32),
        jax.ShapeDtypeStruct((E_ROUTED,), jnp.int32),
        jax.ShapeDtypeStruct((16,), jnp.int32),
    ),
    mesh=plsc.VectorSubcoreMesh(core_axis_name="c", subcore_axis_name="s"),
    compiler_params=pltpu.CompilerParams(needs_layout_passes=False),
    scratch_types=[
        pltpu.VMEM((E_ROUTED, T), jnp.float32),
        pltpu.VMEM((E_ROUTED, T), jnp.float32),
        pltpu.VMEM((TOP_K, T), jnp.int32),
        pltpu.VMEM((TOP_K, T), jnp.float32),
        pltpu.VMEM((E_ROUTED,), jnp.int32),
        pltpu.VMEM((16,), jnp.int32),
    ],
)(_routing_sc_body)


def _moe_body(sched_ref, nact_ref, x_ref, gt_ref,
              swg_ref, swu_ref, swd_ref, rwg_ref, rwu_ref, rwd_ref, out_ref):
    i = pl.program_id(0)
    xb = x_ref[...].astype(jnp.bfloat16)

    @pl.when(i == 0)
    def _shared():
        acc = jnp.zeros((T, D), jnp.float32)
        for e in range(E_SHARED):
            hg = jnp.dot(xb, swg_ref[e].astype(jnp.bfloat16),
                         preferred_element_type=jnp.float32)
            hu = jnp.dot(xb, swu_ref[e].astype(jnp.bfloat16),
                         preferred_element_type=jnp.float32)
            h = hg * jax.lax.logistic(hg) * hu
            acc = acc + jnp.dot(h.astype(jnp.bfloat16),
                                swd_ref[e].astype(jnp.bfloat16),
                                preferred_element_type=jnp.float32)
        out_ref[...] = acc / float(E_SHARED)

    @pl.when(i < nact_ref[0])
    def _routed():
        eid = sched_ref[i]
        hg = jnp.dot(xb, rwg_ref[0].astype(jnp.bfloat16),
                     preferred_element_type=jnp.float32)
        hu = jnp.dot(xb, rwu_ref[0].astype(jnp.bfloat16),
                     preferred_element_type=jnp.float32)
        h = hg * jax.lax.logistic(hg) * hu
        o = jnp.dot(h.astype(jnp.bfloat16), rwd_ref[0].astype(jnp.bfloat16),
                    preferred_element_type=jnp.float32)
        onehot = (jax.lax.broadcasted_iota(jnp.int32, (E_ROUTED, 1), 0)
                  == eid).astype(jnp.float32)
        scale = lax.dot_general(gt_ref[...], onehot, (((0,), (0,)), ((), ())),
                                preferred_element_type=jnp.float32)
        out_ref[...] = out_ref[...] + o * scale


@jax.jit
def kernel(x, router_w, shared_wg, shared_wu, shared_wd,
           routed_wg, routed_wu, routed_wd):
    xf = x.reshape(T, D)

    logits, logits_t = pl.pallas_call(
        _router_body,
        out_shape=(
            jax.ShapeDtypeStruct((T, E_ROUTED), jnp.float32),
            jax.ShapeDtypeStruct((E_ROUTED, T), jnp.float32),
        ),
    )(xf, router_w)

    g_t, idx_t, gates_t, sched, nact_v = _routing_sc(logits_t)
    indices = idx_t.T
    gates = gates_t.T
    nact = nact_v[:1]

    out = pl.pallas_call(
        _moe_body,
        grid_spec=pltpu.PrefetchScalarGridSpec(
            num_scalar_prefetch=2,
            grid=(E_ROUTED,),
            in_specs=[
                pl.BlockSpec((T, D), lambda i, sched, nact: (0, 0)),
                pl.BlockSpec((E_ROUTED, T), lambda i, sched, nact: (0, 0)),
                pl.BlockSpec((E_SHARED, D, H), lambda i, sched, nact: (0, 0, 0)),
                pl.BlockSpec((E_SHARED, D, H), lambda i, sched, nact: (0, 0, 0)),
                pl.BlockSpec((E_SHARED, H, D), lambda i, sched, nact: (0, 0, 0)),
                pl.BlockSpec((1, D, H), lambda i, sched, nact: (sched[i], 0, 0)),
                pl.BlockSpec((1, D, H), lambda i, sched, nact: (sched[i], 0, 0)),
                pl.BlockSpec((1, H, D), lambda i, sched, nact: (sched[i], 0, 0)),
            ],
            out_specs=pl.BlockSpec((T, D), lambda i, sched, nact: (0, 0)),
        ),
        out_shape=jax.ShapeDtypeStruct((T, D), jnp.float32),
        compiler_params=pltpu.CompilerParams(
            dimension_semantics=("arbitrary",),
        ),
    )(sched, nact, xf, g_t,
      shared_wg, shared_wu, shared_wd, routed_wg, routed_wu, routed_wd)

    return out.reshape(B, SEQ, D), logits, indices, gates


# speedup vs baseline: 1.0189x; 1.0189x over previous
import functools

import jax
import jax.numpy as jnp
from jax import lax
from jax.experimental import pallas as pl
from jax.experimental.pallas import tpu as pltpu
from jax.experimental.pallas import tpu_sc as plsc

B, SEQ, D = 32, 1, 1024
E_ROUTED, E_SHARED, H, TOP_K = 64, 2, 512, 2
T = B * SEQ
NEG = -3.0e38


def _router_body(x_ref, rw_ref, logits_ref, logits_t_ref):
    logits_ref[...] = jnp.dot(x_ref[...], rw_ref[...],
                              preferred_element_type=jnp.float32)
    logits_t_ref[...] = lax.dot_general(
        rw_ref[...], x_ref[...], (((0,), (1,)), ((), ())),
        preferred_element_type=jnp.float32)


def _routing_sc_body(lt_hbm, gdt_hbm, idx_hbm, gates_hbm, sched_hbm, nact_hbm,
                     lt_v, gdt_v, idx_v, gates_v, sched_v, nact_v):
    wid = lax.axis_index("s") * 2 + lax.axis_index("c")

    @pl.when(wid == 0)
    def _():
        pltpu.sync_copy(lt_hbm, lt_v)

        iota = lax.iota(jnp.int32, 16)
        zf = jnp.zeros((16,), jnp.float32)
        zi = jnp.zeros((16,), jnp.int32)

        def top2_step(e, c):
            m1a, i1a, m2a, i2a, m1b, i1b, m2b, i2b = c
            ev = jnp.full((16,), e, jnp.int32)
            va = lt_v[e, pl.ds(0, 16)]
            vb = lt_v[e, pl.ds(16, 16)]
            g1 = va > m1a
            g2 = va > m2a
            m2a = jnp.where(g1, m1a, jnp.where(g2, va, m2a))
            i2a = jnp.where(g1, i1a, jnp.where(g2, ev, i2a))
            m1a = jnp.where(g1, va, m1a)
            i1a = jnp.where(g1, ev, i1a)
            g1 = vb > m1b
            g2 = vb > m2b
            m2b = jnp.where(g1, m1b, jnp.where(g2, vb, m2b))
            i2b = jnp.where(g1, i1b, jnp.where(g2, ev, i2b))
            m1b = jnp.where(g1, vb, m1b)
            i1b = jnp.where(g1, ev, i1b)
            return (m1a, i1a, m2a, i2a, m1b, i1b, m2b, i2b)

        neg = jnp.full((16,), NEG, jnp.float32)
        m1a, i1a, m2a, i2a, m1b, i1b, m2b, i2b = lax.fori_loop(
            0, E_ROUTED, top2_step, (neg, zi, neg, zi, neg, zi, neg, zi))

        def den_step(e, c):
            sa, sb = c
            va = lt_v[e, pl.ds(0, 16)]
            vb = lt_v[e, pl.ds(16, 16)]
            return sa + jnp.exp(va - m1a), sb + jnp.exp(vb - m1b)

        sa, sb = lax.fori_loop(0, E_ROUTED, den_step, (zf, zf))
        g1a = 1.0 / sa
        g1b = 1.0 / sb
        g2a = jnp.exp(m2a - m1a) / sa
        g2b = jnp.exp(m2b - m1b) / sb

        idx_v[0, pl.ds(0, 16)] = i1a
        idx_v[0, pl.ds(16, 16)] = i1b
        idx_v[1, pl.ds(0, 16)] = i2a
        idx_v[1, pl.ds(16, 16)] = i2b
        gates_v[0, pl.ds(0, 16)] = g1a
        gates_v[0, pl.ds(16, 16)] = g1b
        gates_v[1, pl.ds(0, 16)] = g2a
        gates_v[1, pl.ds(16, 16)] = g2b

        def expert_step(e, c):
            cnt, last, s0, s1, s2, s3 = c
            ev = jnp.full((16,), e, jnp.int32)
            hit1a = i1a == ev
            hit2a = i2a == ev
            hit1b = i1b == ev
            hit2b = i2b == ev
            ra = jnp.where(hit1a, g1a, 0.0) + jnp.where(hit2a, g2a, 0.0)
            rb = jnp.where(hit1b, g1b, 0.0) + jnp.where(hit2b, g2b, 0.0)
            gdt_v[e, pl.ds(0, 16)] = ra
            gdt_v[e, pl.ds(16, 16)] = rb
            na = (jnp.sum(jnp.where(hit1a | hit2a, 1, 0))
                  + jnp.sum(jnp.where(hit1b | hit2b, 1, 0)))
            active = na > 0
            target = jnp.where(active, cnt, -1)
            s0 = jnp.where(iota == target, e, s0)
            s1 = jnp.where(iota + 16 == target, e, s1)
            s2 = jnp.where(iota + 32 == target, e, s2)
            s3 = jnp.where(iota + 48 == target, e, s3)
            cnt = jnp.where(active, cnt + 1, cnt)
            last = jnp.where(active, e, last)
            return (cnt, last, s0, s1, s2, s3)

        cnt, last, s0, s1, s2, s3 = lax.fori_loop(
            0, E_ROUTED, expert_step,
            (jnp.int32(0), jnp.int32(0), zi, zi, zi, zi))

        lastv = jnp.full((16,), last, jnp.int32)
        sched_v[pl.ds(0, 16)] = jnp.where(iota < cnt, s0, lastv)
        sched_v[pl.ds(16, 16)] = jnp.where(iota + 16 < cnt, s1, lastv)
        sched_v[pl.ds(32, 16)] = jnp.where(iota + 32 < cnt, s2, lastv)
        sched_v[pl.ds(48, 16)] = jnp.where(iota + 48 < cnt, s3, lastv)
        nact_v[...] = jnp.where(iota == 0, cnt, 0)

        pltpu.sync_copy(gdt_v, gdt_hbm)
        pltpu.sync_copy(idx_v, idx_hbm)
        pltpu.sync_copy(gates_v, gates_hbm)
        pltpu.sync_copy(sched_v, sched_hbm)
        pltpu.sync_copy(nact_v, nact_hbm)


_routing_sc = functools.partial(
    pl.kernel,
    out_type=(
        jax.ShapeDtypeStruct((E_ROUTED, T), jnp.float32),
        jax.ShapeDtypeStruct((TOP_K, T), jnp.int32),
        jax.ShapeDtypeStruct((TOP_K, T), jnp.float32),
        jax.ShapeDtypeStruct((E_ROUTED,), jnp.int32),
        jax.ShapeDtypeStruct((16,), jnp.int32),
    ),
    mesh=plsc.VectorSubcoreMesh(core_axis_name="c", subcore_axis_name="s"),
    compiler_params=pltpu.CompilerParams(needs_layout_passes=False),
    scratch_types=[
        pltpu.VMEM((E_ROUTED, T), jnp.float32),
        pltpu.VMEM((E_ROUTED, T), jnp.float32),
        pltpu.VMEM((TOP_K, T), jnp.int32),
        pltpu.VMEM((TOP_K, T), jnp.float32),
        pltpu.VMEM((E_ROUTED,), jnp.int32),
        pltpu.VMEM((16,), jnp.int32),
    ],
)(_routing_sc_body)


def _shared_body(x_ref, swg_ref, swu_ref, swd_ref, out_ref):
    xb = x_ref[...].astype(jnp.bfloat16)
    acc = jnp.zeros((T, D), jnp.float32)
    for e in range(E_SHARED):
        hg = jnp.dot(xb, swg_ref[e].astype(jnp.bfloat16),
                     preferred_element_type=jnp.float32)
        hu = jnp.dot(xb, swu_ref[e].astype(jnp.bfloat16),
                     preferred_element_type=jnp.float32)
        h = hg * jax.lax.logistic(hg) * hu
        acc = acc + jnp.dot(h.astype(jnp.bfloat16),
                            swd_ref[e].astype(jnp.bfloat16),
                            preferred_element_type=jnp.float32)
    out_ref[...] = acc / float(E_SHARED)


def _moe_body(sched_ref, nact_ref, x_ref, gt_ref, shared_ref,
              rwg_ref, rwu_ref, rwd_ref, out_ref):
    i = pl.program_id(0)
    xb = x_ref[...].astype(jnp.bfloat16)

    @pl.when(i == 0)
    def _init():
        out_ref[...] = shared_ref[...]

    @pl.when(i < nact_ref[0])
    def _routed():
        eid = sched_ref[i]
        hg = jnp.dot(xb, rwg_ref[0].astype(jnp.bfloat16),
                     preferred_element_type=jnp.float32)
        hu = jnp.dot(xb, rwu_ref[0].astype(jnp.bfloat16),
                     preferred_element_type=jnp.float32)
        h = hg * jax.lax.logistic(hg) * hu
        o = jnp.dot(h.astype(jnp.bfloat16), rwd_ref[0].astype(jnp.bfloat16),
                    preferred_element_type=jnp.float32)
        onehot = (jax.lax.broadcasted_iota(jnp.int32, (E_ROUTED, 1), 0)
                  == eid).astype(jnp.float32)
        scale = lax.dot_general(gt_ref[...], onehot, (((0,), (0,)), ((), ())),
                                preferred_element_type=jnp.float32)
        out_ref[...] = out_ref[...] + o * scale


@jax.jit
def kernel(x, router_w, shared_wg, shared_wu, shared_wd,
           routed_wg, routed_wu, routed_wd):
    xf = x.reshape(T, D)

    logits, logits_t = pl.pallas_call(
        _router_body,
        out_shape=(
            jax.ShapeDtypeStruct((T, E_ROUTED), jnp.float32),
            jax.ShapeDtypeStruct((E_ROUTED, T), jnp.float32),
        ),
    )(xf, router_w)

    g_t, idx_t, gates_t, sched, nact_v = _routing_sc(logits_t)
    indices = idx_t.T
    gates = gates_t.T
    nact = nact_v[:1]

    shared_out = pl.pallas_call(
        _shared_body,
        out_shape=jax.ShapeDtypeStruct((T, D), jnp.float32),
    )(xf, shared_wg, shared_wu, shared_wd)

    out = pl.pallas_call(
        _moe_body,
        grid_spec=pltpu.PrefetchScalarGridSpec(
            num_scalar_prefetch=2,
            grid=(E_ROUTED,),
            in_specs=[
                pl.BlockSpec((T, D), lambda i, sched, nact: (0, 0)),
                pl.BlockSpec((E_ROUTED, T), lambda i, sched, nact: (0, 0)),
                pl.BlockSpec((T, D), lambda i, sched, nact: (0, 0)),
                pl.BlockSpec((1, D, H), lambda i, sched, nact: (sched[i], 0, 0)),
                pl.BlockSpec((1, D, H), lambda i, sched, nact: (sched[i], 0, 0)),
                pl.BlockSpec((1, H, D), lambda i, sched, nact: (sched[i], 0, 0)),
            ],
            out_specs=pl.BlockSpec((T, D), lambda i, sched, nact: (0, 0)),
        ),
        out_shape=jax.ShapeDtypeStruct((T, D), jnp.float32),
        compiler_params=pltpu.CompilerParams(
            dimension_semantics=("arbitrary",),
        ),
    )(sched, nact, xf, g_t, shared_out,
      routed_wg, routed_wu, routed_wd)

    return out.reshape(B, SEQ, D), logits, indices, gates


# speedup vs baseline: 1.0198x; 1.0009x over previous
import functools

import jax
import jax.numpy as jnp
from jax import lax
from jax.experimental import pallas as pl
from jax.experimental.pallas import tpu as pltpu
from jax.experimental.pallas import tpu_sc as plsc

B, SEQ, D = 32, 1, 1024
E_ROUTED, E_SHARED, H, TOP_K = 64, 2, 512, 2
T = B * SEQ
NEG = -3.0e38


def _router_body(x_ref, rw_ref, logits_ref, logits_t_ref):
    logits_ref[...] = jnp.dot(x_ref[...], rw_ref[...],
                              preferred_element_type=jnp.float32)
    logits_t_ref[...] = lax.dot_general(
        rw_ref[...], x_ref[...], (((0,), (1,)), ((), ())),
        preferred_element_type=jnp.float32)


def _routing_sc_body(lt_hbm, gdt_hbm, idx_hbm, gates_hbm, sched_hbm, nact_hbm,
                     lt_v, gdt_v, idx_v, gates_v, sched_v, nact_v, sem):
    wid = lax.axis_index("s") * 2 + lax.axis_index("c")

    @pl.when(wid == 0)
    def _():
        pltpu.sync_copy(lt_hbm, lt_v)

        iota = lax.iota(jnp.int32, 16)
        zf = jnp.zeros((16,), jnp.float32)
        zi = jnp.zeros((16,), jnp.int32)

        def top2_step(e, c):
            m1a, i1a, m2a, i2a, sa, m1b, i1b, m2b, i2b, sb = c
            ev = jnp.full((16,), e, jnp.int32)
            va = lt_v[e, pl.ds(0, 16)]
            vb = lt_v[e, pl.ds(16, 16)]
            g1 = va > m1a
            g2 = va > m2a
            mn = jnp.where(g1, va, m1a)
            sa = sa * jnp.exp(m1a - mn) + jnp.exp(va - mn)
            m2a = jnp.where(g1, m1a, jnp.where(g2, va, m2a))
            i2a = jnp.where(g1, i1a, jnp.where(g2, ev, i2a))
            m1a = mn
            i1a = jnp.where(g1, ev, i1a)
            g1 = vb > m1b
            g2 = vb > m2b
            mn = jnp.where(g1, vb, m1b)
            sb = sb * jnp.exp(m1b - mn) + jnp.exp(vb - mn)
            m2b = jnp.where(g1, m1b, jnp.where(g2, vb, m2b))
            i2b = jnp.where(g1, i1b, jnp.where(g2, ev, i2b))
            m1b = mn
            i1b = jnp.where(g1, ev, i1b)
            return (m1a, i1a, m2a, i2a, sa, m1b, i1b, m2b, i2b, sb)

        neg = jnp.full((16,), NEG, jnp.float32)
        m1a, i1a, m2a, i2a, sa, m1b, i1b, m2b, i2b, sb = lax.fori_loop(
            0, E_ROUTED, top2_step,
            (neg, zi, neg, zi, zf, neg, zi, neg, zi, zf), unroll=8)
        g1a = 1.0 / sa
        g1b = 1.0 / sb
        g2a = jnp.exp(m2a - m1a) / sa
        g2b = jnp.exp(m2b - m1b) / sb

        idx_v[0, pl.ds(0, 16)] = i1a
        idx_v[0, pl.ds(16, 16)] = i1b
        idx_v[1, pl.ds(0, 16)] = i2a
        idx_v[1, pl.ds(16, 16)] = i2b
        gates_v[0, pl.ds(0, 16)] = g1a
        gates_v[0, pl.ds(16, 16)] = g1b
        gates_v[1, pl.ds(0, 16)] = g2a
        gates_v[1, pl.ds(16, 16)] = g2b

        def expert_step(e, c):
            cnt, last, s0, s1, s2, s3 = c
            ev = jnp.full((16,), e, jnp.int32)
            hit1a = i1a == ev
            hit2a = i2a == ev
            hit1b = i1b == ev
            hit2b = i2b == ev
            ra = jnp.where(hit1a, g1a, 0.0) + jnp.where(hit2a, g2a, 0.0)
            rb = jnp.where(hit1b, g1b, 0.0) + jnp.where(hit2b, g2b, 0.0)
            gdt_v[e, pl.ds(0, 16)] = ra
            gdt_v[e, pl.ds(16, 16)] = rb
            na = (jnp.sum(jnp.where(hit1a | hit2a, 1, 0))
                  + jnp.sum(jnp.where(hit1b | hit2b, 1, 0)))
            active = na > 0
            target = jnp.where(active, cnt, -1)
            s0 = jnp.where(iota == target, e, s0)
            s1 = jnp.where(iota + 16 == target, e, s1)
            s2 = jnp.where(iota + 32 == target, e, s2)
            s3 = jnp.where(iota + 48 == target, e, s3)
            cnt = jnp.where(active, cnt + 1, cnt)
            last = jnp.where(active, e, last)
            return (cnt, last, s0, s1, s2, s3)

        cnt, last, s0, s1, s2, s3 = lax.fori_loop(
            0, E_ROUTED, expert_step,
            (jnp.int32(0), jnp.int32(0), zi, zi, zi, zi), unroll=8)

        lastv = jnp.full((16,), last, jnp.int32)
        sched_v[pl.ds(0, 16)] = jnp.where(iota < cnt, s0, lastv)
        sched_v[pl.ds(16, 16)] = jnp.where(iota + 16 < cnt, s1, lastv)
        sched_v[pl.ds(32, 16)] = jnp.where(iota + 32 < cnt, s2, lastv)
        sched_v[pl.ds(48, 16)] = jnp.where(iota + 48 < cnt, s3, lastv)
        nact_v[...] = jnp.where(iota == 0, cnt, 0)

        c1 = pltpu.async_copy(gdt_v, gdt_hbm, sem)
        c2 = pltpu.async_copy(idx_v, idx_hbm, sem)
        c3 = pltpu.async_copy(gates_v, gates_hbm, sem)
        c4 = pltpu.async_copy(sched_v, sched_hbm, sem)
        c5 = pltpu.async_copy(nact_v, nact_hbm, sem)
        c1.wait()
        c2.wait()
        c3.wait()
        c4.wait()
        c5.wait()


_routing_sc = functools.partial(
    pl.kernel,
    out_type=(
        jax.ShapeDtypeStruct((E_ROUTED, T), jnp.float32),
        jax.ShapeDtypeStruct((TOP_K, T), jnp.int32),
        jax.ShapeDtypeStruct((TOP_K, T), jnp.float32),
        jax.ShapeDtypeStruct((E_ROUTED,), jnp.int32),
        jax.ShapeDtypeStruct((16,), jnp.int32),
    ),
    mesh=plsc.VectorSubcoreMesh(core_axis_name="c", subcore_axis_name="s"),
    compiler_params=pltpu.CompilerParams(needs_layout_passes=False),
    scratch_types=[
        pltpu.VMEM((E_ROUTED, T), jnp.float32),
        pltpu.VMEM((E_ROUTED, T), jnp.float32),
        pltpu.VMEM((TOP_K, T), jnp.int32),
        pltpu.VMEM((TOP_K, T), jnp.float32),
        pltpu.VMEM((E_ROUTED,), jnp.int32),
        pltpu.VMEM((16,), jnp.int32),
        pltpu.SemaphoreType.DMA,
    ],
)(_routing_sc_body)


def _shared_body(x_ref, swg_ref, swu_ref, swd_ref, out_ref):
    xb = x_ref[...].astype(jnp.bfloat16)
    acc = jnp.zeros((T, D), jnp.float32)
    for e in range(E_SHARED):
        hg = jnp.dot(xb, swg_ref[e].astype(jnp.bfloat16),
                     preferred_element_type=jnp.float32)
        hu = jnp.dot(xb, swu_ref[e].astype(jnp.bfloat16),
                     preferred_element_type=jnp.float32)
        h = hg * jax.lax.logistic(hg) * hu
        acc = acc + jnp.dot(h.astype(jnp.bfloat16),
                            swd_ref[e].astype(jnp.bfloat16),
                            preferred_element_type=jnp.float32)
    out_ref[...] = acc / float(E_SHARED)


def _moe_body(sched_ref, nact_ref, x_ref, gt_ref, shared_ref,
              rwg_ref, rwu_ref, rwd_ref, out_ref):
    i = pl.program_id(0)
    xb = x_ref[...].astype(jnp.bfloat16)

    @pl.when(i == 0)
    def _init():
        out_ref[...] = shared_ref[...]

    @pl.when(i < nact_ref[0])
    def _routed():
        eid = sched_ref[i]
        hg = jnp.dot(xb, rwg_ref[0].astype(jnp.bfloat16),
                     preferred_element_type=jnp.float32)
        hu = jnp.dot(xb, rwu_ref[0].astype(jnp.bfloat16),
                     preferred_element_type=jnp.float32)
        h = hg * jax.lax.logistic(hg) * hu
        o = jnp.dot(h.astype(jnp.bfloat16), rwd_ref[0].astype(jnp.bfloat16),
                    preferred_element_type=jnp.float32)
        onehot = (jax.lax.broadcasted_iota(jnp.int32, (E_ROUTED, 1), 0)
                  == eid).astype(jnp.float32)
        scale = lax.dot_general(gt_ref[...], onehot, (((0,), (0,)), ((), ())),
                                preferred_element_type=jnp.float32)
        out_ref[...] = out_ref[...] + o * scale


@jax.jit
def kernel(x, router_w, shared_wg, shared_wu, shared_wd,
           routed_wg, routed_wu, routed_wd):
    xf = x.reshape(T, D)

    logits, logits_t = pl.pallas_call(
        _router_body,
        out_shape=(
            jax.ShapeDtypeStruct((T, E_ROUTED), jnp.float32),
            jax.ShapeDtypeStruct((E_ROUTED, T), jnp.float32),
        ),
    )(xf, router_w)

    shared_out = pl.pallas_call(
        _shared_body,
        out_shape=jax.ShapeDtypeStruct((T, D), jnp.float32),
    )(xf, shared_wg, shared_wu, shared_wd)

    g_t, idx_t, gates_t, sched, nact_v = _routing_sc(logits_t)
    indices = idx_t.T
    gates = gates_t.T
    nact = nact_v[:1]

    out = pl.pallas_call(
        _moe_body,
        grid_spec=pltpu.PrefetchScalarGridSpec(
            num_scalar_prefetch=2,
            grid=(E_ROUTED,),
            in_specs=[
                pl.BlockSpec((T, D), lambda i, sched, nact: (0, 0)),
                pl.BlockSpec((E_ROUTED, T), lambda i, sched, nact: (0, 0)),
                pl.BlockSpec((T, D), lambda i, sched, nact: (0, 0)),
                pl.BlockSpec((1, D, H), lambda i, sched, nact: (sched[i], 0, 0)),
                pl.BlockSpec((1, D, H), lambda i, sched, nact: (sched[i], 0, 0)),
                pl.BlockSpec((1, H, D), lambda i, sched, nact: (sched[i], 0, 0)),
            ],
            out_specs=pl.BlockSpec((T, D), lambda i, sched, nact: (0, 0)),
        ),
        out_shape=jax.ShapeDtypeStruct((T, D), jnp.float32),
        compiler_params=pltpu.CompilerParams(
            dimension_semantics=("arbitrary",),
        ),
    )(sched, nact, xf, g_t, shared_out,
      routed_wg, routed_wu, routed_wd)

    return out.reshape(B, SEQ, D), logits, indices, gates
